# baseline (speedup 1.0000x reference)
import functools

import jax
import jax.numpy as jnp
from jax import lax
from jax.experimental import pallas as pl
from jax.experimental.pallas import tpu as pltpu
from jax.experimental.pallas import tpu_sc as plsc

_N = 50000
_E = 800000
_IN = 3
_HID = 8
_DIM = _IN + _HID
_NITER = 3

_NC = 2
_NS = 16
_NW = _NC * _NS
_C = 128
_CHUNKS = 200
_EPT = _C * _CHUNKS
_EPAD = _EPT * _NW
_ROWS_E = _EPAD // _C
_NP = 50048
_ZR = 184
_RPT = _NP // _NS
_NZC = _RPT // _ZR


def _sc_mesh():
  return plsc.VectorSubcoreMesh(core_axis_name="c", subcore_axis_name="s",
                                num_cores=_NC, num_subcores=_NS)


def _sc_msg_body(t_hbm, col_hbm, row_hbm, consts_hbm,
                 mi_out, mo_out,
                 mi_acc, mo_acc, col_a, row_a,
                 gc0, gc1, gr0, gr1, mi0, mi1, mo0, mo1,
                 zbuf, cbuf,
                 sem_g0, sem_g1, sem_s0, sem_s1, sem_i0, sem_i1, sem_z):
  cid = lax.axis_index("c")
  sid = lax.axis_index("s")
  wid = cid * _NS + sid

  pltpu.sync_copy(consts_hbm, cbuf)
  be1p = cbuf[0]
  tw2 = cbuf[1]
  be2k = cbuf[2]

  def zrow(i, carry):
    zbuf[i] = jnp.zeros((16,), jnp.float32)
    return carry
  lax.fori_loop(0, _ZR, zrow, 0)

  def zmsg(i, carry):
    mi0[i] = jnp.zeros((16,), jnp.float32)
    mi1[i] = jnp.zeros((16,), jnp.float32)
    mo0[i] = jnp.zeros((16,), jnp.float32)
    mo1[i] = jnp.zeros((16,), jnp.float32)
    return carry
  lax.fori_loop(0, _C, zmsg, 0)

  base_r = sid * _RPT

  def zissue(i, carry):
    off = base_r + i * _ZR
    pltpu.async_copy(zbuf, mi_acc.at[pl.ds(off, _ZR)], sem_z)
    pltpu.async_copy(zbuf, mo_acc.at[pl.ds(off, _ZR)], sem_z)
    return carry
  lax.fori_loop(0, _NZC, zissue, 0)

  def zdrain(i, carry):
    pltpu.make_async_copy(zbuf, mi_acc.at[pl.ds(base_r, _ZR)], sem_z).wait()
    pltpu.make_async_copy(zbuf, mo_acc.at[pl.ds(base_r, _ZR)], sem_z).wait()
    return carry
  lax.fori_loop(0, _NZC, zdrain, 0)
  plsc.subcore_barrier()

  gcs = (gc0, gc1)
  grs = (gr0, gr1)
  mis = (mi0, mi1)
  mos = (mo0, mo1)
  sgs = (sem_g0, sem_g1)
  sss = (sem_s0, sem_s1)
  sis = (sem_i0, sem_i1)
  iota16 = lax.iota(jnp.int32, 16)
  tile_base_e = wid * _EPT
  tb_row = wid * _CHUNKS

  pltpu.sync_copy(col_hbm.at[tb_row], col_a.at[0])
  pltpu.sync_copy(row_hbm.at[tb_row], row_a.at[0])
  pltpu.async_copy(col_hbm.at[tb_row + 1], col_a.at[1], sem_i1)
  pltpu.async_copy(row_hbm.at[tb_row + 1], row_a.at[1], sem_i1)
  pltpu.async_copy(t_hbm.at[col_a.at[0]], gc0, sem_g0)
  pltpu.async_copy(t_hbm.at[row_a.at[0]], gr0, sem_g0)

  def outer(k4, carry):
    for u in range(4):
      k = k4 * 4 + u
      b = u % 2
      nb = 1 - b
      sl = u
      nsl = (u + 1) % 4
      fsl = (u + 2) % 4
      gcb = gcs[b]
      grb = grs[b]
      mib = mis[b]
      mob = mos[b]

      pltpu.make_async_copy(t_hbm.at[col_a.at[sl]], gcb, sgs[b]).wait()
      pltpu.make_async_copy(t_hbm.at[row_a.at[sl]], grb, sgs[b]).wait()

      @pl.when(k + 2 < _CHUNKS)
      def _issue_idx():
        pltpu.async_copy(col_hbm.at[tb_row + k + 2], col_a.at[fsl], sis[b])
        pltpu.async_copy(row_hbm.at[tb_row + k + 2], row_a.at[fsl], sis[b])

      @pl.when(k + 1 < _CHUNKS)
      def _issue_gather():
        pltpu.make_async_copy(col_hbm.at[tb_row + k + 1], col_a.at[nsl],
                              sis[nb]).wait()
        pltpu.make_async_copy(row_hbm.at[tb_row + k + 1], row_a.at[nsl],
                              sis[nb]).wait()
        pltpu.async_copy(t_hbm.at[col_a.at[nsl]], gcs[nb], sgs[nb])
        pltpu.async_copy(t_hbm.at[row_a.at[nsl]], grs[nb], sgs[nb])

      def grp(g, c2):
        rows = g * 16 + iota16
        s = be2k
        for j in range(8):
          a = plsc.load_gather(gcb, [rows, jnp.full((16,), j, jnp.int32)])
          bb = plsc.load_gather(grb, [rows, jnp.full((16,), 8 + j, jnp.int32)])
          q = jnp.exp(a + bb) + 1.0
          s = s - tw2[j] / q
        ev = 1.0 / (1.0 + jnp.exp(-s))
        gid = tile_base_e + k * _C + g * 16 + iota16
        ev = jnp.where(gid < _E, ev, 0.0)
        for f in range(_DIM):
          cf = jnp.full((16,), 16 + f, jnp.int32)
          ff = jnp.full((16,), f, jnp.int32)
          xr = plsc.load_gather(grb, [rows, cf])
          plsc.store_scatter(mib, [rows, ff], ev * xr)
          xcv = plsc.load_gather(gcb, [rows, cf])
          plsc.store_scatter(mob, [rows, ff], ev * xcv)
        return c2
      lax.fori_loop(0, _C // 16, grp, 0)

      @pl.when(k >= 1)
      def _wait_prev_scatter():
        pltpu.make_async_copy(mis[nb], mi_acc.at[col_a.at[sl]], sss[nb]).wait()
        pltpu.make_async_copy(mos[nb], mo_acc.at[row_a.at[sl]], sss[nb]).wait()
      pltpu.async_copy(mib, mi_acc.at[col_a.at[sl]], sss[b], add=True)
      pltpu.async_copy(mob, mo_acc.at[row_a.at[sl]], sss[b], add=True)
    return carry
  lax.fori_loop(0, _CHUNKS // 4, outer, 0)

  pltpu.make_async_copy(mis[1], mi_acc.at[col_a.at[0]], sss[1]).wait()
  pltpu.make_async_copy(mos[1], mo_acc.at[row_a.at[0]], sss[1]).wait()
  plsc.subcore_barrier()

  def dissue(i, carry):
    off = base_r + i * _ZR
    pltpu.async_copy(mi_acc.at[pl.ds(off, _ZR)],
                     mi_out.at[cid, pl.ds(off, _ZR)], sem_z)
    pltpu.async_copy(mo_acc.at[pl.ds(off, _ZR)],
                     mo_out.at[cid, pl.ds(off, _ZR)], sem_z)
    return carry
  lax.fori_loop(0, _NZC, dissue, 0)

  def ddrain(i, carry):
    pltpu.make_async_copy(mi_acc.at[pl.ds(base_r, _ZR)],
                          mi_out.at[cid, pl.ds(base_r, _ZR)], sem_z).wait()
    pltpu.make_async_copy(mo_acc.at[pl.ds(base_r, _ZR)],
                          mo_out.at[cid, pl.ds(base_r, _ZR)], sem_z).wait()
    return carry
  lax.fori_loop(0, _NZC, ddrain, 0)


def _sc_msg(t, col, row, consts):
  f = functools.partial(
      pl.kernel,
      out_type=(jax.ShapeDtypeStruct((_NC, _NP, 16), jnp.float32),
                jax.ShapeDtypeStruct((_NC, _NP, 16), jnp.float32)),
      mesh=_sc_mesh(),
      compiler_params=pltpu.CompilerParams(needs_layout_passes=False,
                                           use_tc_tiling_on_sc=False),
      scratch_types=[
          pltpu.VMEM_SHARED((_NP, 16), jnp.float32),
          pltpu.VMEM_SHARED((_NP, 16), jnp.float32),
          pltpu.VMEM((4, _C), jnp.int32),
          pltpu.VMEM((4, _C), jnp.int32),
          pltpu.VMEM((_C, 32), jnp.float32),
          pltpu.VMEM((_C, 32), jnp.float32),
          pltpu.VMEM((_C, 32), jnp.float32),
          pltpu.VMEM((_C, 32), jnp.float32),
          pltpu.VMEM((_C, 16), jnp.float32),
          pltpu.VMEM((_C, 16), jnp.float32),
          pltpu.VMEM((_C, 16), jnp.float32),
          pltpu.VMEM((_C, 16), jnp.float32),
          pltpu.VMEM((_ZR, 16), jnp.float32),
          pltpu.VMEM((4, 16), jnp.float32),
          pltpu.SemaphoreType.DMA,
          pltpu.SemaphoreType.DMA,
          pltpu.SemaphoreType.DMA,
          pltpu.SemaphoreType.DMA,
          pltpu.SemaphoreType.DMA,
          pltpu.SemaphoreType.DMA,
          pltpu.SemaphoreType.DMA,
      ],
  )(_sc_msg_body)
  return f(t, col, row, consts)


def _sc_final_body(t_hbm, col_hbm, row_hbm, consts_hbm, e_out,
                   col_a, row_a, gc0, gc1, gr0, gr1, ebuf, cbuf,
                   sem_g0, sem_g1, sem_i0, sem_i1):
  cid = lax.axis_index("c")
  sid = lax.axis_index("s")
  wid = cid * _NS + sid

  pltpu.sync_copy(consts_hbm, cbuf)
  be1p = cbuf[0]
  tw2 = cbuf[1]
  be2k = cbuf[2]

  tb_row = wid * _CHUNKS

  gcs = (gc0, gc1)
  grs = (gr0, gr1)
  sgs = (sem_g0, sem_g1)
  sis = (sem_i0, sem_i1)
  iota16 = lax.iota(jnp.int32, 16)

  pltpu.sync_copy(col_hbm.at[tb_row], col_a.at[0])
  pltpu.sync_copy(row_hbm.at[tb_row], row_a.at[0])
  pltpu.async_copy(col_hbm.at[tb_row + 1], col_a.at[1], sem_i1)
  pltpu.async_copy(row_hbm.at[tb_row + 1], row_a.at[1], sem_i1)
  pltpu.async_copy(t_hbm.at[col_a.at[0]], gc0, sem_g0)
  pltpu.async_copy(t_hbm.at[row_a.at[0]], gr0, sem_g0)

  def outer(k4, carry):
    for u in range(4):
      k = k4 * 4 + u
      b = u % 2
      nb = 1 - b
      sl = u
      nsl = (u + 1) % 4
      fsl = (u + 2) % 4
      gcb = gcs[b]
      grb = grs[b]

      pltpu.make_async_copy(t_hbm.at[col_a.at[sl]], gcb, sgs[b]).wait()
      pltpu.make_async_copy(t_hbm.at[row_a.at[sl]], grb, sgs[b]).wait()

      @pl.when(k + 2 < _CHUNKS)
      def _issue_idx():
        pltpu.async_copy(col_hbm.at[tb_row + k + 2], col_a.at[fsl], sis[b])
        pltpu.async_copy(row_hbm.at[tb_row + k + 2], row_a.at[fsl], sis[b])

      @pl.when(k + 1 < _CHUNKS)
      def _issue_gather():
        pltpu.make_async_copy(col_hbm.at[tb_row + k + 1], col_a.at[nsl],
                              sis[nb]).wait()
        pltpu.make_async_copy(row_hbm.at[tb_row + k + 1], row_a.at[nsl],
                              sis[nb]).wait()
        pltpu.async_copy(t_hbm.at[col_a.at[nsl]], gcs[nb], sgs[nb])
        pltpu.async_copy(t_hbm.at[row_a.at[nsl]], grs[nb], sgs[nb])

      def grp(g, c2):
        rows = g * 16 + iota16
        s = be2k
        for j in range(8):
          a = plsc.load_gather(gcb, [rows, jnp.full((16,), j, jnp.int32)])
          bb = plsc.load_gather(grb, [rows, jnp.full((16,), 8 + j, jnp.int32)])
          q = jnp.exp(a + bb) + 1.0
          s = s - tw2[j] / q
        ev = 1.0 / (1.0 + jnp.exp(-s))
        off = pl.multiple_of(g * 16, 16)
        ebuf[k, pl.ds(off, 16)] = ev
        return c2
      lax.fori_loop(0, _C // 16, grp, 0)
    return carry
  lax.fori_loop(0, _CHUNKS // 4, outer, 0)

  pltpu.sync_copy(ebuf, e_out.at[pl.ds(tb_row, _CHUNKS)])


def _sc_final(t, col, row, consts):
  f = functools.partial(
      pl.kernel,
      out_type=jax.ShapeDtypeStruct((_ROWS_E, _C), jnp.float32),
      mesh=_sc_mesh(),
      compiler_params=pltpu.CompilerParams(needs_layout_passes=False,
                                           use_tc_tiling_on_sc=False),
      scratch_types=[
          pltpu.VMEM((4, _C), jnp.int32),
          pltpu.VMEM((4, _C), jnp.int32),
          pltpu.VMEM((_C, 16), jnp.float32),
          pltpu.VMEM((_C, 16), jnp.float32),
          pltpu.VMEM((_C, 16), jnp.float32),
          pltpu.VMEM((_C, 16), jnp.float32),
          pltpu.VMEM((_CHUNKS, _C), jnp.float32),
          pltpu.VMEM((4, 16), jnp.float32),
          pltpu.SemaphoreType.DMA,
          pltpu.SemaphoreType.DMA,
          pltpu.SemaphoreType.DMA,
          pltpu.SemaphoreType.DMA,
      ],
  )(_sc_final_body)
  return f(t, col, row, consts)


_BN = 2000


def _tc_init_body(x_ref, w1, b1, wa, wb, b1e, t_ref):
  xb = x_ref[...]
  h = jnp.tanh(jnp.dot(xb, w1[...], preferred_element_type=jnp.float32)
               + b1[...])
  xc = jnp.concatenate([h, xb], axis=1)
  pa = 2.0 * jnp.dot(xc, wa[...], preferred_element_type=jnp.float32) + b1e[...]
  pb = 2.0 * jnp.dot(xc, wb[...], preferred_element_type=jnp.float32) + b1e[...]
  z = jnp.zeros((xb.shape[0], 32 - 2 * _HID - _DIM), jnp.float32)
  t_ref[...] = jnp.concatenate([pa, pb, xc, z], axis=1)


def _tc_init(x, w1, b1, wa, wb, b1e):
  return pl.pallas_call(
      _tc_init_body,
      grid=(_N // _BN,),
      in_specs=[
          pl.BlockSpec((_BN, _IN), lambda i: (i, 0)),
          pl.BlockSpec((_IN, _HID), lambda i: (0, 0)),
          pl.BlockSpec((1, _HID), lambda i: (0, 0)),
          pl.BlockSpec((_DIM, _HID), lambda i: (0, 0)),
          pl.BlockSpec((_DIM, _HID), lambda i: (0, 0)),
          pl.BlockSpec((1, _HID), lambda i: (0, 0)),
      ],
      out_specs=pl.BlockSpec((_BN, 32), lambda i: (i, 0)),
      out_shape=jax.ShapeDtypeStruct((_N, 32), jnp.float32),
  )(x, w1, b1, wa, wb, b1e)


def _tc_iter_body(mi2, mo2, t_ref, wn1, bn1, wn2, bn2, wa, wb, b1e, to_ref):
  mi = (mi2[0] + mi2[1])[:, :_DIM]
  mo = (mo2[0] + mo2[1])[:, :_DIM]
  xc = t_ref[:, 16:16 + _DIM]
  m = jnp.concatenate([mi, mo, xc], axis=1)
  h1 = jnp.tanh(jnp.dot(m, wn1[...], preferred_element_type=jnp.float32)
                + bn1[...])
  hn = jnp.tanh(jnp.dot(h1, wn2[...], preferred_element_type=jnp.float32)
                + bn2[...])
  xcn = jnp.concatenate([hn, xc[:, _HID:_DIM]], axis=1)
  pa = 2.0 * jnp.dot(xcn, wa[...], preferred_element_type=jnp.float32) + b1e[...]
  pb = 2.0 * jnp.dot(xcn, wb[...], preferred_element_type=jnp.float32) + b1e[...]
  z = jnp.zeros((xcn.shape[0], 32 - 2 * _HID - _DIM), jnp.float32)
  to_ref[...] = jnp.concatenate([pa, pb, xcn, z], axis=1)


def _tc_iter(mi2, mo2, t, wn1, bn1, wn2, bn2, wa, wb, b1e):
  return pl.pallas_call(
      _tc_iter_body,
      grid=(_N // _BN,),
      in_specs=[
          pl.BlockSpec((_NC, _BN, 16), lambda i: (0, i, 0)),
          pl.BlockSpec((_NC, _BN, 16), lambda i: (0, i, 0)),
          pl.BlockSpec((_BN, 32), lambda i: (i, 0)),
          pl.BlockSpec((3 * _DIM, _HID), lambda i: (0, 0)),
          pl.BlockSpec((1, _HID), lambda i: (0, 0)),
          pl.BlockSpec((_HID, _HID), lambda i: (0, 0)),
          pl.BlockSpec((1, _HID), lambda i: (0, 0)),
          pl.BlockSpec((_DIM, _HID), lambda i: (0, 0)),
          pl.BlockSpec((_DIM, _HID), lambda i: (0, 0)),
          pl.BlockSpec((1, _HID), lambda i: (0, 0)),
      ],
      out_specs=pl.BlockSpec((_BN, 32), lambda i: (i, 0)),
      out_shape=jax.ShapeDtypeStruct((_N, 32), jnp.float32),
  )(mi2, mo2, t, wn1, bn1, wn2, bn2, wa, wb, b1e)


def kernel(x, edge_index, W1, b1, We1, be1, We2, be2, Wn1, bn1, Wn2, bn2):
  row = edge_index[0].astype(jnp.int32)
  col = edge_index[1].astype(jnp.int32)
  pad = _EPAD - _E
  colp = jnp.concatenate([col, jnp.zeros((pad,), jnp.int32)])
  colp = colp.reshape(_ROWS_E, _C)
  rowp = jnp.concatenate([row, jnp.zeros((pad,), jnp.int32)])
  rowp = rowp.reshape(_ROWS_E, _C)

  wa = We1[:_DIM]
  wb = We1[_DIM:]
  zero8 = jnp.zeros((_HID,), jnp.float32)
  be1p = jnp.concatenate([be1, zero8])
  tw2 = jnp.concatenate([2.0 * We2[:, 0], zero8])
  be2k = jnp.full((16,), be2[0] + jnp.sum(We2[:, 0]), jnp.float32)
  consts = jnp.stack([be1p, tw2, be2k, jnp.zeros((16,), jnp.float32)])

  b1r = b1.reshape(1, _HID)
  bn1r = bn1.reshape(1, _HID)
  bn2r = bn2.reshape(1, _HID)

  b1e = be1.reshape(1, _HID)
  t = _tc_init(x, W1, b1r, wa, wb, b1e)
  for _ in range(_NITER):
    mi2, mo2 = _sc_msg(t, colp, rowp, consts)
    t = _tc_iter(mi2, mo2, t, Wn1, bn1r, Wn2, bn2r, wa, wb, b1e)
  e = _sc_final(t[:, :16], colp, rowp, consts)
  return e.reshape(_EPAD)[:_E]


# speedup vs baseline: 1.4889x; 1.4889x over previous
import functools

import jax
import jax.numpy as jnp
from jax import lax
from jax.experimental import pallas as pl
from jax.experimental.pallas import tpu as pltpu
from jax.experimental.pallas import tpu_sc as plsc

_N = 50000
_E = 800000
_IN = 3
_HID = 8
_DIM = _IN + _HID
_NITER = 3

_NC = 2
_NS = 16
_NW = _NC * _NS
_C = 128
_CHUNKS = 200
_EPT = _C * _CHUNKS
_EPAD = _EPT * _NW
_ROWS_E = _EPAD // _C
_NP = 50048
_ZR = 184
_RPT = _NP // _NS
_NZC = _RPT // _ZR


def _sc_mesh():
  return plsc.VectorSubcoreMesh(core_axis_name="c", subcore_axis_name="s",
                                num_cores=_NC, num_subcores=_NS)


def _sc_msg_body(tc_hbm, tr_hbm, col_hbm, row_hbm, consts_hbm, z8_hbm,
                 mi_out, mo_out,
                 mi_acc, mo_acc, col_a, row_a,
                 gc0, gc1, gr0, gr1, mi0, mi1, mo0, mo1,
                 zbuf, cbuf,
                 sem_g0, sem_g1, sem_s0, sem_s1, sem_i0, sem_i1, sem_z):
  cid = lax.axis_index("c")
  sid = lax.axis_index("s")
  wid = cid * _NS + sid

  pltpu.sync_copy(consts_hbm, cbuf)
  tw2 = cbuf[1]
  be2k = cbuf[2]

  pltpu.sync_copy(z8_hbm, zbuf)

  base_r = sid * _RPT

  def zissue(i, carry):
    off = base_r + i * _ZR
    pltpu.async_copy(zbuf, mi_acc.at[pl.ds(off, _ZR)], sem_z)
    pltpu.async_copy(zbuf, mo_acc.at[pl.ds(off, _ZR)], sem_z)
    return carry
  lax.fori_loop(0, _NZC, zissue, 0)

  def zdrain(i, carry):
    pltpu.make_async_copy(zbuf, mi_acc.at[pl.ds(base_r, _ZR)], sem_z).wait()
    pltpu.make_async_copy(zbuf, mo_acc.at[pl.ds(base_r, _ZR)], sem_z).wait()
    return carry
  lax.fori_loop(0, _NZC, zdrain, 0)
  plsc.subcore_barrier()

  gcs = (gc0, gc1)
  grs = (gr0, gr1)
  mis = (mi0, mi1)
  mos = (mo0, mo1)
  sgs = (sem_g0, sem_g1)
  sss = (sem_s0, sem_s1)
  sis = (sem_i0, sem_i1)
  iota16 = lax.iota(jnp.int32, 16)
  tile_base_e = wid * _EPT
  tb_row = wid * _CHUNKS

  pltpu.sync_copy(col_hbm.at[tb_row], col_a.at[0])
  pltpu.sync_copy(row_hbm.at[tb_row], row_a.at[0])
  pltpu.async_copy(col_hbm.at[tb_row + 1], col_a.at[1], sem_i1)
  pltpu.async_copy(row_hbm.at[tb_row + 1], row_a.at[1], sem_i1)
  pltpu.async_copy(tc_hbm.at[col_a.at[0]], gc0, sem_g0)
  pltpu.async_copy(tr_hbm.at[row_a.at[0]], gr0, sem_g0)

  def outer(k4, carry):
    for u in range(4):
      k = k4 * 4 + u
      b = u % 2
      nb = 1 - b
      sl = u
      nsl = (u + 1) % 4
      fsl = (u + 2) % 4
      gcb = gcs[b]
      grb = grs[b]
      mib = mis[b]
      mob = mos[b]

      pltpu.make_async_copy(tc_hbm.at[col_a.at[sl]], gcb, sgs[b]).wait()
      pltpu.make_async_copy(tr_hbm.at[row_a.at[sl]], grb, sgs[b]).wait()

      @pl.when(k + 2 < _CHUNKS)
      def _issue_idx():
        pltpu.async_copy(col_hbm.at[tb_row + k + 2], col_a.at[fsl], sis[b])
        pltpu.async_copy(row_hbm.at[tb_row + k + 2], row_a.at[fsl], sis[b])

      @pl.when(k + 1 < _CHUNKS)
      def _issue_gather():
        pltpu.make_async_copy(col_hbm.at[tb_row + k + 1], col_a.at[nsl],
                              sis[nb]).wait()
        pltpu.make_async_copy(row_hbm.at[tb_row + k + 1], row_a.at[nsl],
                              sis[nb]).wait()
        pltpu.async_copy(tc_hbm.at[col_a.at[nsl]], gcs[nb], sgs[nb])
        pltpu.async_copy(tr_hbm.at[row_a.at[nsl]], grs[nb], sgs[nb])

      def grp(g, c2):
        rows = g * 16 + iota16
        s = be2k
        for j in range(8):
          a = plsc.load_gather(gcb, [rows, jnp.full((16,), j, jnp.int32)])
          bb = plsc.load_gather(grb, [rows, jnp.full((16,), j, jnp.int32)])
          q = jnp.exp(a + bb) + 1.0
          s = s - tw2[j] / q
        ev = 1.0 / (1.0 + jnp.exp(-s))
        gid = tile_base_e + k * _C + g * 16 + iota16
        ev = jnp.where(gid < _E, ev, 0.0)
        for f in range(_HID):
          pf = jnp.full((16,), 8 + f, jnp.int32)
          ff = jnp.full((16,), f, jnp.int32)
          xr = plsc.load_gather(grb, [rows, pf])
          plsc.store_scatter(mib, [rows, ff], ev * xr)
          xcv = plsc.load_gather(gcb, [rows, pf])
          plsc.store_scatter(mob, [rows, ff], ev * xcv)
        return c2
      lax.fori_loop(0, _C // 16, grp, 0)

      @pl.when(k >= 1)
      def _wait_prev_scatter():
        pltpu.make_async_copy(mis[nb], mi_acc.at[col_a.at[sl]], sss[nb]).wait()
        pltpu.make_async_copy(mos[nb], mo_acc.at[row_a.at[sl]], sss[nb]).wait()
      pltpu.async_copy(mib, mi_acc.at[col_a.at[sl]], sss[b], add=True)
      pltpu.async_copy(mob, mo_acc.at[row_a.at[sl]], sss[b], add=True)
    return carry
  lax.fori_loop(0, _CHUNKS // 4, outer, 0)

  pltpu.make_async_copy(mis[1], mi_acc.at[col_a.at[0]], sss[1]).wait()
  pltpu.make_async_copy(mos[1], mo_acc.at[row_a.at[0]], sss[1]).wait()
  plsc.subcore_barrier()

  def dissue(i, carry):
    off = base_r + i * _ZR
    pltpu.async_copy(mi_acc.at[pl.ds(off, _ZR)],
                     mi_out.at[cid, pl.ds(off, _ZR)], sem_z)
    pltpu.async_copy(mo_acc.at[pl.ds(off, _ZR)],
                     mo_out.at[cid, pl.ds(off, _ZR)], sem_z)
    return carry
  lax.fori_loop(0, _NZC, dissue, 0)

  def ddrain(i, carry):
    pltpu.make_async_copy(mi_acc.at[pl.ds(base_r, _ZR)],
                          mi_out.at[cid, pl.ds(base_r, _ZR)], sem_z).wait()
    pltpu.make_async_copy(mo_acc.at[pl.ds(base_r, _ZR)],
                          mo_out.at[cid, pl.ds(base_r, _ZR)], sem_z).wait()
    return carry
  lax.fori_loop(0, _NZC, ddrain, 0)


def _sc_msg(tc, tr, col, row, consts, z8):
  f = functools.partial(
      pl.kernel,
      out_type=(jax.ShapeDtypeStruct((_NC, _NP, _HID), jnp.float32),
                jax.ShapeDtypeStruct((_NC, _NP, _HID), jnp.float32)),
      mesh=_sc_mesh(),
      compiler_params=pltpu.CompilerParams(needs_layout_passes=False,
                                           use_tc_tiling_on_sc=False),
      scratch_types=[
          pltpu.VMEM_SHARED((_NP, _HID), jnp.float32),
          pltpu.VMEM_SHARED((_NP, _HID), jnp.float32),
          pltpu.VMEM((4, _C), jnp.int32),
          pltpu.VMEM((4, _C), jnp.int32),
          pltpu.VMEM((_C, 16), jnp.float32),
          pltpu.VMEM((_C, 16), jnp.float32),
          pltpu.VMEM((_C, 16), jnp.float32),
          pltpu.VMEM((_C, 16), jnp.float32),
          pltpu.VMEM((_C, _HID), jnp.float32),
          pltpu.VMEM((_C, _HID), jnp.float32),
          pltpu.VMEM((_C, _HID), jnp.float32),
          pltpu.VMEM((_C, _HID), jnp.float32),
          pltpu.VMEM((_ZR, _HID), jnp.float32),
          pltpu.VMEM((4, 16), jnp.float32),
          pltpu.SemaphoreType.DMA,
          pltpu.SemaphoreType.DMA,
          pltpu.SemaphoreType.DMA,
          pltpu.SemaphoreType.DMA,
          pltpu.SemaphoreType.DMA,
          pltpu.SemaphoreType.DMA,
          pltpu.SemaphoreType.DMA,
      ],
  )(_sc_msg_body)
  return f(tc, tr, col, row, consts, z8)


def _sc_final_body(tc_hbm, tr_hbm, col_hbm, row_hbm, consts_hbm, e_out,
                   col_a, row_a, gc0, gc1, gr0, gr1, ebuf, cbuf,
                   sem_g0, sem_g1, sem_i0, sem_i1):
  cid = lax.axis_index("c")
  sid = lax.axis_index("s")
  wid = cid * _NS + sid

  pltpu.sync_copy(consts_hbm, cbuf)
  tw2 = cbuf[1]
  be2k = cbuf[2]

  tb_row = wid * _CHUNKS

  gcs = (gc0, gc1)
  grs = (gr0, gr1)
  sgs = (sem_g0, sem_g1)
  sis = (sem_i0, sem_i1)
  iota16 = lax.iota(jnp.int32, 16)

  pltpu.sync_copy(col_hbm.at[tb_row], col_a.at[0])
  pltpu.sync_copy(row_hbm.at[tb_row], row_a.at[0])
  pltpu.async_copy(col_hbm.at[tb_row + 1], col_a.at[1], sem_i1)
  pltpu.async_copy(row_hbm.at[tb_row + 1], row_a.at[1], sem_i1)
  pltpu.async_copy(tc_hbm.at[col_a.at[0]], gc0, sem_g0)
  pltpu.async_copy(tr_hbm.at[row_a.at[0]], gr0, sem_g0)

  def outer(k4, carry):
    for u in range(4):
      k = k4 * 4 + u
      b = u % 2
      nb = 1 - b
      sl = u
      nsl = (u + 1) % 4
      fsl = (u + 2) % 4
      gcb = gcs[b]
      grb = grs[b]

      pltpu.make_async_copy(tc_hbm.at[col_a.at[sl]], gcb, sgs[b]).wait()
      pltpu.make_async_copy(tr_hbm.at[row_a.at[sl]], grb, sgs[b]).wait()

      @pl.when(k + 2 < _CHUNKS)
      def _issue_idx():
        pltpu.async_copy(col_hbm.at[tb_row + k + 2], col_a.at[fsl], sis[b])
        pltpu.async_copy(row_hbm.at[tb_row + k + 2], row_a.at[fsl], sis[b])

      @pl.when(k + 1 < _CHUNKS)
      def _issue_gather():
        pltpu.make_async_copy(col_hbm.at[tb_row + k + 1], col_a.at[nsl],
                              sis[nb]).wait()
        pltpu.make_async_copy(row_hbm.at[tb_row + k + 1], row_a.at[nsl],
                              sis[nb]).wait()
        pltpu.async_copy(tc_hbm.at[col_a.at[nsl]], gcs[nb], sgs[nb])
        pltpu.async_copy(tr_hbm.at[row_a.at[nsl]], grs[nb], sgs[nb])

      def grp(g, c2):
        rows = g * 16 + iota16
        s = be2k
        for j in range(8):
          a = plsc.load_gather(gcb, [rows, jnp.full((16,), j, jnp.int32)])
          bb = plsc.load_gather(grb, [rows, jnp.full((16,), j, jnp.int32)])
          q = jnp.exp(a + bb) + 1.0
          s = s - tw2[j] / q
        ev = 1.0 / (1.0 + jnp.exp(-s))
        off = pl.multiple_of(g * 16, 16)
        ebuf[k, pl.ds(off, 16)] = ev
        return c2
      lax.fori_loop(0, _C // 16, grp, 0)
    return carry
  lax.fori_loop(0, _CHUNKS // 4, outer, 0)

  pltpu.sync_copy(ebuf, e_out.at[pl.ds(tb_row, _CHUNKS)])


def _sc_final(tc, tr, col, row, consts):
  f = functools.partial(
      pl.kernel,
      out_type=jax.ShapeDtypeStruct((_ROWS_E, _C), jnp.float32),
      mesh=_sc_mesh(),
      compiler_params=pltpu.CompilerParams(needs_layout_passes=False,
                                           use_tc_tiling_on_sc=False),
      scratch_types=[
          pltpu.VMEM((4, _C), jnp.int32),
          pltpu.VMEM((4, _C), jnp.int32),
          pltpu.VMEM((_C, 16), jnp.float32),
          pltpu.VMEM((_C, 16), jnp.float32),
          pltpu.VMEM((_C, 16), jnp.float32),
          pltpu.VMEM((_C, 16), jnp.float32),
          pltpu.VMEM((_CHUNKS, _C), jnp.float32),
          pltpu.VMEM((4, 16), jnp.float32),
          pltpu.SemaphoreType.DMA,
          pltpu.SemaphoreType.DMA,
          pltpu.SemaphoreType.DMA,
          pltpu.SemaphoreType.DMA,
      ],
  )(_sc_final_body)
  return f(tc, tr, col, row, consts)


_BN = 2000


def _tables(xc, wa, wb, wn1a, wn1b, b1e):
  u = 2.0 * jnp.dot(xc, wa, preferred_element_type=jnp.float32) + b1e
  v = 2.0 * jnp.dot(xc, wb, preferred_element_type=jnp.float32) + b1e
  p1 = jnp.dot(xc, wn1a, preferred_element_type=jnp.float32)
  p2 = jnp.dot(xc, wn1b, preferred_element_type=jnp.float32)
  tcv = jnp.concatenate([u, p2], axis=1)
  trv = jnp.concatenate([v, p1], axis=1)
  return tcv, trv


def _tc_init_body(x_ref, w1, b1, wa, wb, wn1a, wn1b, b1e,
                  tc_ref, tr_ref, xc_ref):
  xb = x_ref[...]
  h = jnp.tanh(jnp.dot(xb, w1[...], preferred_element_type=jnp.float32)
               + b1[...])
  xc = jnp.concatenate([h, xb], axis=1)
  tcv, trv = _tables(xc, wa[...], wb[...], wn1a[...], wn1b[...], b1e[...])
  tc_ref[...] = tcv
  tr_ref[...] = trv
  z = jnp.zeros((xb.shape[0], 16 - _DIM), jnp.float32)
  xc_ref[...] = jnp.concatenate([xc, z], axis=1)


def _tc_init(x, w1, b1, wa, wb, wn1a, wn1b, b1e):
  wspec = pl.BlockSpec((_DIM, _HID), lambda i: (0, 0))
  bspec = pl.BlockSpec((1, _HID), lambda i: (0, 0))
  return pl.pallas_call(
      _tc_init_body,
      grid=(_N // _BN,),
      in_specs=[
          pl.BlockSpec((_BN, _IN), lambda i: (i, 0)),
          pl.BlockSpec((_IN, _HID), lambda i: (0, 0)),
          bspec, wspec, wspec, wspec, wspec, bspec,
      ],
      out_specs=[pl.BlockSpec((_BN, 16), lambda i: (i, 0))] * 3,
      out_shape=[jax.ShapeDtypeStruct((_N, 16), jnp.float32)] * 3,
  )(x, w1, b1, wa, wb, wn1a, wn1b, b1e)


def _tc_iter_body(mi2, mo2, xc_in, wn1c, bn1, wn2, bn2, wa, wb, wn1a, wn1b,
                  b1e, tc_ref, tr_ref, xc_ref):
  mi = mi2[0] + mi2[1]
  mo = mo2[0] + mo2[1]
  xc = xc_in[:, :_DIM]
  h1 = jnp.tanh(mi + mo
                + jnp.dot(xc, wn1c[...], preferred_element_type=jnp.float32)
                + bn1[...])
  hn = jnp.tanh(jnp.dot(h1, wn2[...], preferred_element_type=jnp.float32)
                + bn2[...])
  xcn = jnp.concatenate([hn, xc[:, _HID:_DIM]], axis=1)
  tcv, trv = _tables(xcn, wa[...], wb[...], wn1a[...], wn1b[...], b1e[...])
  tc_ref[...] = tcv
  tr_ref[...] = trv
  z = jnp.zeros((xcn.shape[0], 16 - _DIM), jnp.float32)
  xc_ref[...] = jnp.concatenate([xcn, z], axis=1)


def _tc_iter(mi2, mo2, xca, wn1c, bn1, wn2, bn2, wa, wb, wn1a, wn1b, b1e):
  wspec = pl.BlockSpec((_DIM, _HID), lambda i: (0, 0))
  bspec = pl.BlockSpec((1, _HID), lambda i: (0, 0))
  return pl.pallas_call(
      _tc_iter_body,
      grid=(_N // _BN,),
      in_specs=[
          pl.BlockSpec((_NC, _BN, _HID), lambda i: (0, i, 0)),
          pl.BlockSpec((_NC, _BN, _HID), lambda i: (0, i, 0)),
          pl.BlockSpec((_BN, 16), lambda i: (i, 0)),
          wspec, bspec,
          pl.BlockSpec((_HID, _HID), lambda i: (0, 0)),
          bspec, wspec, wspec, wspec, wspec, bspec,
      ],
      out_specs=[pl.BlockSpec((_BN, 16), lambda i: (i, 0))] * 3,
      out_shape=[jax.ShapeDtypeStruct((_N, 16), jnp.float32)] * 3,
  )(mi2, mo2, xca, wn1c, bn1, wn2, bn2, wa, wb, wn1a, wn1b, b1e)


def kernel(x, edge_index, W1, b1, We1, be1, We2, be2, Wn1, bn1, Wn2, bn2):
  row = edge_index[0].astype(jnp.int32)
  col = edge_index[1].astype(jnp.int32)
  pad = _EPAD - _E
  colp = jnp.concatenate([col, jnp.zeros((pad,), jnp.int32)])
  colp = colp.reshape(_ROWS_E, _C)
  rowp = jnp.concatenate([row, jnp.zeros((pad,), jnp.int32)])
  rowp = rowp.reshape(_ROWS_E, _C)

  wa = We1[:_DIM]
  wb = We1[_DIM:]
  wn1a = Wn1[:_DIM]
  wn1b = Wn1[_DIM:2 * _DIM]
  wn1c = Wn1[2 * _DIM:]
  zero8 = jnp.zeros((_HID,), jnp.float32)
  be1p = jnp.concatenate([be1, zero8])
  tw2 = jnp.concatenate([2.0 * We2[:, 0], zero8])
  be2k = jnp.full((16,), be2[0] + jnp.sum(We2[:, 0]), jnp.float32)
  consts = jnp.stack([be1p, tw2, be2k, jnp.zeros((16,), jnp.float32)])
  z8 = jnp.zeros((_ZR, _HID), jnp.float32)

  b1r = b1.reshape(1, _HID)
  bn1r = bn1.reshape(1, _HID)
  bn2r = bn2.reshape(1, _HID)
  b1e = be1.reshape(1, _HID)

  tc, tr, xca = _tc_init(x, W1, b1r, wa, wb, wn1a, wn1b, b1e)
  for _ in range(_NITER):
    mi2, mo2 = _sc_msg(tc, tr, colp, rowp, consts, z8)
    tc, tr, xca = _tc_iter(mi2, mo2, xca, wn1c, bn1r, Wn2, bn2r,
                           wa, wb, wn1a, wn1b, b1e)
  e = _sc_final(tc, tr, colp, rowp, consts)
  return e.reshape(_EPAD)[:_E]


# speedup vs baseline: 1.5356x; 1.0314x over previous
import functools

import jax
import jax.numpy as jnp
from jax import lax
from jax.experimental import pallas as pl
from jax.experimental.pallas import tpu as pltpu
from jax.experimental.pallas import tpu_sc as plsc

_N = 50000
_E = 800000
_IN = 3
_HID = 8
_DIM = _IN + _HID
_NITER = 3

_NC = 2
_NS = 16
_NW = _NC * _NS
_C = 256
_CS = 128
_CHUNKS = 100
_EPT = _C * _CHUNKS
_EPAD = _EPT * _NW
_ROWS_E = _EPAD // _C
_NP = 50048
_ZR = 184
_RPT = _NP // _NS
_NZC = _RPT // _ZR


def _sc_mesh():
  return plsc.VectorSubcoreMesh(core_axis_name="c", subcore_axis_name="s",
                                num_cores=_NC, num_subcores=_NS)


def _sc_msg_body(tc_hbm, tr_hbm, col_hbm, row_hbm, consts_hbm, z8_hbm,
                 mi_out, mo_out,
                 mi_acc, mo_acc, col_a, row_a,
                 gc0, gc1, gr0, gr1, mi0, mi1, mo0, mo1,
                 zbuf, cbuf,
                 sem_g0, sem_g1, sem_s0, sem_s1, sem_i0, sem_i1, sem_z):
  cid = lax.axis_index("c")
  sid = lax.axis_index("s")
  wid = cid * _NS + sid

  pltpu.sync_copy(consts_hbm, cbuf)
  tw2 = cbuf[1]
  be2k = cbuf[2]

  pltpu.sync_copy(z8_hbm, zbuf)

  base_r = sid * _RPT

  def zissue(i, carry):
    off = base_r + i * _ZR
    pltpu.async_copy(zbuf, mi_acc.at[pl.ds(off, _ZR)], sem_z)
    pltpu.async_copy(zbuf, mo_acc.at[pl.ds(off, _ZR)], sem_z)
    return carry
  lax.fori_loop(0, _NZC, zissue, 0)

  def zdrain(i, carry):
    pltpu.make_async_copy(zbuf, mi_acc.at[pl.ds(base_r, _ZR)], sem_z).wait()
    pltpu.make_async_copy(zbuf, mo_acc.at[pl.ds(base_r, _ZR)], sem_z).wait()
    return carry
  lax.fori_loop(0, _NZC, zdrain, 0)
  plsc.subcore_barrier()

  gcs = (gc0, gc1)
  grs = (gr0, gr1)
  mis = (mi0, mi1)
  mos = (mo0, mo1)
  sgs = (sem_g0, sem_g1)
  sss = (sem_s0, sem_s1)
  sis = (sem_i0, sem_i1)
  iota16 = lax.iota(jnp.int32, 16)
  tile_base_e = wid * _EPT
  tb_row = wid * _CHUNKS

  pltpu.sync_copy(col_hbm.at[tb_row], col_a.at[0])
  pltpu.sync_copy(row_hbm.at[tb_row], row_a.at[0])
  pltpu.async_copy(col_hbm.at[tb_row + 1], col_a.at[1], sem_i1)
  pltpu.async_copy(row_hbm.at[tb_row + 1], row_a.at[1], sem_i1)
  for _j in range(2):
    pltpu.async_copy(tc_hbm.at[col_a.at[0, _j]],
                     gc0.at[pl.ds(_j * _CS, _CS)], sem_g0)
    pltpu.async_copy(tr_hbm.at[row_a.at[0, _j]],
                     gr0.at[pl.ds(_j * _CS, _CS)], sem_g0)

  def outer(k4, carry):
    for u in range(4):
      k = k4 * 4 + u
      b = u % 2
      nb = 1 - b
      sl = u
      nsl = (u + 1) % 4
      fsl = (u + 2) % 4
      gcb = gcs[b]
      grb = grs[b]
      mib = mis[b]
      mob = mos[b]

      for _j in range(2):
        pltpu.make_async_copy(tc_hbm.at[col_a.at[sl, _j]],
                              gcb.at[pl.ds(_j * _CS, _CS)], sgs[b]).wait()
        pltpu.make_async_copy(tr_hbm.at[row_a.at[sl, _j]],
                              grb.at[pl.ds(_j * _CS, _CS)], sgs[b]).wait()

      @pl.when(k + 2 < _CHUNKS)
      def _issue_idx():
        pltpu.async_copy(col_hbm.at[tb_row + k + 2], col_a.at[fsl], sis[b])
        pltpu.async_copy(row_hbm.at[tb_row + k + 2], row_a.at[fsl], sis[b])

      @pl.when(k + 1 < _CHUNKS)
      def _issue_gather():
        pltpu.make_async_copy(col_hbm.at[tb_row + k + 1], col_a.at[nsl],
                              sis[nb]).wait()
        pltpu.make_async_copy(row_hbm.at[tb_row + k + 1], row_a.at[nsl],
                              sis[nb]).wait()
        for _j in range(2):
          pltpu.async_copy(tc_hbm.at[col_a.at[nsl, _j]],
                           gcs[nb].at[pl.ds(_j * _CS, _CS)], sgs[nb])
          pltpu.async_copy(tr_hbm.at[row_a.at[nsl, _j]],
                           grs[nb].at[pl.ds(_j * _CS, _CS)], sgs[nb])

      def grp(g, c2):
        rows = g * 16 + iota16
        s = be2k
        for j in range(8):
          a = plsc.load_gather(gcb, [rows, jnp.full((16,), j, jnp.int32)])
          bb = plsc.load_gather(grb, [rows, jnp.full((16,), j, jnp.int32)])
          q = jnp.exp(a + bb) + 1.0
          s = s - tw2[j] / q
        ev = 1.0 / (1.0 + jnp.exp(-s))
        gid = tile_base_e + k * _C + g * 16 + iota16
        ev = jnp.where(gid < _E, ev, 0.0)
        for f in range(_HID):
          pf = jnp.full((16,), 8 + f, jnp.int32)
          ff = jnp.full((16,), f, jnp.int32)
          xr = plsc.load_gather(grb, [rows, pf])
          plsc.store_scatter(mib, [rows, ff], ev * xr)
          xcv = plsc.load_gather(gcb, [rows, pf])
          plsc.store_scatter(mob, [rows, ff], ev * xcv)
        return c2
      lax.fori_loop(0, _C // 16, grp, 0)

      @pl.when(k >= 1)
      def _wait_prev_scatter():
        for _j in range(2):
          pltpu.make_async_copy(mis[nb].at[pl.ds(_j * _CS, _CS)],
                                mi_acc.at[col_a.at[sl, _j]], sss[nb]).wait()
          pltpu.make_async_copy(mos[nb].at[pl.ds(_j * _CS, _CS)],
                                mo_acc.at[row_a.at[sl, _j]], sss[nb]).wait()
      for _j in range(2):
        pltpu.async_copy(mib.at[pl.ds(_j * _CS, _CS)],
                         mi_acc.at[col_a.at[sl, _j]], sss[b], add=True)
        pltpu.async_copy(mob.at[pl.ds(_j * _CS, _CS)],
                         mo_acc.at[row_a.at[sl, _j]], sss[b], add=True)
    return carry
  lax.fori_loop(0, _CHUNKS // 4, outer, 0)

  for _j in range(2):
    pltpu.make_async_copy(mis[1].at[pl.ds(_j * _CS, _CS)],
                          mi_acc.at[col_a.at[0, _j]], sss[1]).wait()
    pltpu.make_async_copy(mos[1].at[pl.ds(_j * _CS, _CS)],
                          mo_acc.at[row_a.at[0, _j]], sss[1]).wait()
  plsc.subcore_barrier()

  def dissue(i, carry):
    off = base_r + i * _ZR
    pltpu.async_copy(mi_acc.at[pl.ds(off, _ZR)],
                     mi_out.at[cid, pl.ds(off, _ZR)], sem_z)
    pltpu.async_copy(mo_acc.at[pl.ds(off, _ZR)],
                     mo_out.at[cid, pl.ds(off, _ZR)], sem_z)
    return carry
  lax.fori_loop(0, _NZC, dissue, 0)

  def ddrain(i, carry):
    pltpu.make_async_copy(mi_acc.at[pl.ds(base_r, _ZR)],
                          mi_out.at[cid, pl.ds(base_r, _ZR)], sem_z).wait()
    pltpu.make_async_copy(mo_acc.at[pl.ds(base_r, _ZR)],
                          mo_out.at[cid, pl.ds(base_r, _ZR)], sem_z).wait()
    return carry
  lax.fori_loop(0, _NZC, ddrain, 0)


def _sc_msg(tc, tr, col, row, consts, z8):
  f = functools.partial(
      pl.kernel,
      out_type=(jax.ShapeDtypeStruct((_NC, _NP, _HID), jnp.float32),
                jax.ShapeDtypeStruct((_NC, _NP, _HID), jnp.float32)),
      mesh=_sc_mesh(),
      compiler_params=pltpu.CompilerParams(needs_layout_passes=False,
                                           use_tc_tiling_on_sc=False),
      scratch_types=[
          pltpu.VMEM_SHARED((_NP, _HID), jnp.float32),
          pltpu.VMEM_SHARED((_NP, _HID), jnp.float32),
          pltpu.VMEM((4, 2, _CS), jnp.int32),
          pltpu.VMEM((4, 2, _CS), jnp.int32),
          pltpu.VMEM((_C, 16), jnp.float32),
          pltpu.VMEM((_C, 16), jnp.float32),
          pltpu.VMEM((_C, 16), jnp.float32),
          pltpu.VMEM((_C, 16), jnp.float32),
          pltpu.VMEM((_C, _HID), jnp.float32),
          pltpu.VMEM((_C, _HID), jnp.float32),
          pltpu.VMEM((_C, _HID), jnp.float32),
          pltpu.VMEM((_C, _HID), jnp.float32),
          pltpu.VMEM((_ZR, _HID), jnp.float32),
          pltpu.VMEM((4, 16), jnp.float32),
          pltpu.SemaphoreType.DMA,
          pltpu.SemaphoreType.DMA,
          pltpu.SemaphoreType.DMA,
          pltpu.SemaphoreType.DMA,
          pltpu.SemaphoreType.DMA,
          pltpu.SemaphoreType.DMA,
          pltpu.SemaphoreType.DMA,
      ],
  )(_sc_msg_body)
  return f(tc, tr, col, row, consts, z8)


def _sc_final_body(tc_hbm, tr_hbm, col_hbm, row_hbm, consts_hbm, e_out,
                   col_a, row_a, gc0, gc1, gr0, gr1, ebuf, cbuf,
                   sem_g0, sem_g1, sem_i0, sem_i1):
  cid = lax.axis_index("c")
  sid = lax.axis_index("s")
  wid = cid * _NS + sid

  pltpu.sync_copy(consts_hbm, cbuf)
  tw2 = cbuf[1]
  be2k = cbuf[2]

  tb_row = wid * _CHUNKS

  gcs = (gc0, gc1)
  grs = (gr0, gr1)
  sgs = (sem_g0, sem_g1)
  sis = (sem_i0, sem_i1)
  iota16 = lax.iota(jnp.int32, 16)

  pltpu.sync_copy(col_hbm.at[tb_row], col_a.at[0])
  pltpu.sync_copy(row_hbm.at[tb_row], row_a.at[0])
  pltpu.async_copy(col_hbm.at[tb_row + 1], col_a.at[1], sem_i1)
  pltpu.async_copy(row_hbm.at[tb_row + 1], row_a.at[1], sem_i1)
  for _j in range(2):
    pltpu.async_copy(tc_hbm.at[col_a.at[0, _j]],
                     gc0.at[pl.ds(_j * _CS, _CS)], sem_g0)
    pltpu.async_copy(tr_hbm.at[row_a.at[0, _j]],
                     gr0.at[pl.ds(_j * _CS, _CS)], sem_g0)

  def outer(k4, carry):
    for u in range(4):
      k = k4 * 4 + u
      b = u % 2
      nb = 1 - b
      sl = u
      nsl = (u + 1) % 4
      fsl = (u + 2) % 4
      gcb = gcs[b]
      grb = grs[b]

      for _j in range(2):
        pltpu.make_async_copy(tc_hbm.at[col_a.at[sl, _j]],
                              gcb.at[pl.ds(_j * _CS, _CS)], sgs[b]).wait()
        pltpu.make_async_copy(tr_hbm.at[row_a.at[sl, _j]],
                              grb.at[pl.ds(_j * _CS, _CS)], sgs[b]).wait()

      @pl.when(k + 2 < _CHUNKS)
      def _issue_idx():
        pltpu.async_copy(col_hbm.at[tb_row + k + 2], col_a.at[fsl], sis[b])
        pltpu.async_copy(row_hbm.at[tb_row + k + 2], row_a.at[fsl], sis[b])

      @pl.when(k + 1 < _CHUNKS)
      def _issue_gather():
        pltpu.make_async_copy(col_hbm.at[tb_row + k + 1], col_a.at[nsl],
                              sis[nb]).wait()
        pltpu.make_async_copy(row_hbm.at[tb_row + k + 1], row_a.at[nsl],
                              sis[nb]).wait()
        for _j in range(2):
          pltpu.async_copy(tc_hbm.at[col_a.at[nsl, _j]],
                           gcs[nb].at[pl.ds(_j * _CS, _CS)], sgs[nb])
          pltpu.async_copy(tr_hbm.at[row_a.at[nsl, _j]],
                           grs[nb].at[pl.ds(_j * _CS, _CS)], sgs[nb])

      def grp(g, c2):
        rows = g * 16 + iota16
        s = be2k
        for j in range(8):
          a = plsc.load_gather(gcb, [rows, jnp.full((16,), j, jnp.int32)])
          bb = plsc.load_gather(grb, [rows, jnp.full((16,), j, jnp.int32)])
          q = jnp.exp(a + bb) + 1.0
          s = s - tw2[j] / q
        ev = 1.0 / (1.0 + jnp.exp(-s))
        off = pl.multiple_of(g * 16, 16)
        ebuf[k, pl.ds(off, 16)] = ev
        return c2
      lax.fori_loop(0, _C // 16, grp, 0)
    return carry
  lax.fori_loop(0, _CHUNKS // 4, outer, 0)

  pltpu.sync_copy(ebuf, e_out.at[pl.ds(tb_row, _CHUNKS)])


def _sc_final(tc, tr, col, row, consts):
  f = functools.partial(
      pl.kernel,
      out_type=jax.ShapeDtypeStruct((_ROWS_E, _C), jnp.float32),
      mesh=_sc_mesh(),
      compiler_params=pltpu.CompilerParams(needs_layout_passes=False,
                                           use_tc_tiling_on_sc=False),
      scratch_types=[
          pltpu.VMEM((4, 2, _CS), jnp.int32),
          pltpu.VMEM((4, 2, _CS), jnp.int32),
          pltpu.VMEM((_C, 16), jnp.float32),
          pltpu.VMEM((_C, 16), jnp.float32),
          pltpu.VMEM((_C, 16), jnp.float32),
          pltpu.VMEM((_C, 16), jnp.float32),
          pltpu.VMEM((_CHUNKS, _C), jnp.float32),
          pltpu.VMEM((4, 16), jnp.float32),
          pltpu.SemaphoreType.DMA,
          pltpu.SemaphoreType.DMA,
          pltpu.SemaphoreType.DMA,
          pltpu.SemaphoreType.DMA,
      ],
  )(_sc_final_body)
  return f(tc, tr, col, row, consts)


_BN = 2000


def _tables(xc, wa, wb, wn1a, wn1b, b1e):
  u = 2.0 * jnp.dot(xc, wa, preferred_element_type=jnp.float32) + b1e
  v = 2.0 * jnp.dot(xc, wb, preferred_element_type=jnp.float32) + b1e
  p1 = jnp.dot(xc, wn1a, preferred_element_type=jnp.float32)
  p2 = jnp.dot(xc, wn1b, preferred_element_type=jnp.float32)
  tcv = jnp.concatenate([u, p2], axis=1)
  trv = jnp.concatenate([v, p1], axis=1)
  return tcv, trv


def _tc_init_body(x_ref, w1, b1, wa, wb, wn1a, wn1b, b1e,
                  tc_ref, tr_ref, xc_ref):
  xb = x_ref[...]
  h = jnp.tanh(jnp.dot(xb, w1[...], preferred_element_type=jnp.float32)
               + b1[...])
  xc = jnp.concatenate([h, xb], axis=1)
  tcv, trv = _tables(xc, wa[...], wb[...], wn1a[...], wn1b[...], b1e[...])
  tc_ref[...] = tcv
  tr_ref[...] = trv
  z = jnp.zeros((xb.shape[0], 16 - _DIM), jnp.float32)
  xc_ref[...] = jnp.concatenate([xc, z], axis=1)


def _tc_init(x, w1, b1, wa, wb, wn1a, wn1b, b1e):
  wspec = pl.BlockSpec((_DIM, _HID), lambda i: (0, 0))
  bspec = pl.BlockSpec((1, _HID), lambda i: (0, 0))
  return pl.pallas_call(
      _tc_init_body,
      grid=(_N // _BN,),
      in_specs=[
          pl.BlockSpec((_BN, _IN), lambda i: (i, 0)),
          pl.BlockSpec((_IN, _HID), lambda i: (0, 0)),
          bspec, wspec, wspec, wspec, wspec, bspec,
      ],
      out_specs=[pl.BlockSpec((_BN, 16), lambda i: (i, 0))] * 3,
      out_shape=[jax.ShapeDtypeStruct((_N, 16), jnp.float32)] * 3,
  )(x, w1, b1, wa, wb, wn1a, wn1b, b1e)


def _tc_iter_body(mi2, mo2, xc_in, wn1c, bn1, wn2, bn2, wa, wb, wn1a, wn1b,
                  b1e, tc_ref, tr_ref, xc_ref):
  mi = mi2[0] + mi2[1]
  mo = mo2[0] + mo2[1]
  xc = xc_in[:, :_DIM]
  h1 = jnp.tanh(mi + mo
                + jnp.dot(xc, wn1c[...], preferred_element_type=jnp.float32)
                + bn1[...])
  hn = jnp.tanh(jnp.dot(h1, wn2[...], preferred_element_type=jnp.float32)
                + bn2[...])
  xcn = jnp.concatenate([hn, xc[:, _HID:_DIM]], axis=1)
  tcv, trv = _tables(xcn, wa[...], wb[...], wn1a[...], wn1b[...], b1e[...])
  tc_ref[...] = tcv
  tr_ref[...] = trv
  z = jnp.zeros((xcn.shape[0], 16 - _DIM), jnp.float32)
  xc_ref[...] = jnp.concatenate([xcn, z], axis=1)


def _tc_iter(mi2, mo2, xca, wn1c, bn1, wn2, bn2, wa, wb, wn1a, wn1b, b1e):
  wspec = pl.BlockSpec((_DIM, _HID), lambda i: (0, 0))
  bspec = pl.BlockSpec((1, _HID), lambda i: (0, 0))
  return pl.pallas_call(
      _tc_iter_body,
      grid=(_N // _BN,),
      in_specs=[
          pl.BlockSpec((_NC, _BN, _HID), lambda i: (0, i, 0)),
          pl.BlockSpec((_NC, _BN, _HID), lambda i: (0, i, 0)),
          pl.BlockSpec((_BN, 16), lambda i: (i, 0)),
          wspec, bspec,
          pl.BlockSpec((_HID, _HID), lambda i: (0, 0)),
          bspec, wspec, wspec, wspec, wspec, bspec,
      ],
      out_specs=[pl.BlockSpec((_BN, 16), lambda i: (i, 0))] * 3,
      out_shape=[jax.ShapeDtypeStruct((_N, 16), jnp.float32)] * 3,
  )(mi2, mo2, xca, wn1c, bn1, wn2, bn2, wa, wb, wn1a, wn1b, b1e)


def kernel(x, edge_index, W1, b1, We1, be1, We2, be2, Wn1, bn1, Wn2, bn2):
  row = edge_index[0].astype(jnp.int32)
  col = edge_index[1].astype(jnp.int32)
  pad = _EPAD - _E
  colp = jnp.concatenate([col, jnp.zeros((pad,), jnp.int32)])
  colp = colp.reshape(_ROWS_E, 2, _CS)
  rowp = jnp.concatenate([row, jnp.zeros((pad,), jnp.int32)])
  rowp = rowp.reshape(_ROWS_E, 2, _CS)

  wa = We1[:_DIM]
  wb = We1[_DIM:]
  wn1a = Wn1[:_DIM]
  wn1b = Wn1[_DIM:2 * _DIM]
  wn1c = Wn1[2 * _DIM:]
  zero8 = jnp.zeros((_HID,), jnp.float32)
  be1p = jnp.concatenate([be1, zero8])
  tw2 = jnp.concatenate([2.0 * We2[:, 0], zero8])
  be2k = jnp.full((16,), be2[0] + jnp.sum(We2[:, 0]), jnp.float32)
  consts = jnp.stack([be1p, tw2, be2k, jnp.zeros((16,), jnp.float32)])
  z8 = jnp.zeros((_ZR, _HID), jnp.float32)

  b1r = b1.reshape(1, _HID)
  bn1r = bn1.reshape(1, _HID)
  bn2r = bn2.reshape(1, _HID)
  b1e = be1.reshape(1, _HID)

  tc, tr, xca = _tc_init(x, W1, b1r, wa, wb, wn1a, wn1b, b1e)
  for _ in range(_NITER):
    mi2, mo2 = _sc_msg(tc, tr, colp, rowp, consts, z8)
    tc, tr, xca = _tc_iter(mi2, mo2, xca, wn1c, bn1r, Wn2, bn2r,
                           wa, wb, wn1a, wn1b, b1e)
  e = _sc_final(tc, tr, colp, rowp, consts)
  return e.reshape(_EPAD)[:_E]


# speedup vs baseline: 1.5673x; 1.0207x over previous
import functools

import jax
import jax.numpy as jnp
from jax import lax
from jax.experimental import pallas as pl
from jax.experimental.pallas import tpu as pltpu
from jax.experimental.pallas import tpu_sc as plsc

_N = 50000
_E = 800000
_IN = 3
_HID = 8
_DIM = _IN + _HID
_NITER = 3

_NC = 2
_NS = 16
_NW = _NC * _NS
_C = 256
_CS = 128
_CHUNKS = 100
_EPT = _C * _CHUNKS
_EPAD = _EPT * _NW
_ROWS_E = _EPAD // _C
_NP = 50048
_ZR = 184
_RPT = _NP // _NS
_NZC = _RPT // _ZR


def _sc_mesh():
  return plsc.VectorSubcoreMesh(core_axis_name="c", subcore_axis_name="s",
                                num_cores=_NC, num_subcores=_NS)


def _sc_msg_body(tc_hbm, tr_hbm, col_hbm, row_hbm, consts_hbm, z8_hbm,
                 mi_out, mo_out,
                 mi_acc, mo_acc, col_a, row_a,
                 gc0, gc1, gr0, gr1, mi0, mi1, mo0, mo1,
                 zbuf, cbuf,
                 sem_g0, sem_g1, sem_s0, sem_s1, sem_i0, sem_i1, sem_z):
  cid = lax.axis_index("c")
  sid = lax.axis_index("s")
  wid = cid * _NS + sid

  pltpu.sync_copy(consts_hbm, cbuf)
  tw2 = cbuf[1]
  be2k = cbuf[2]

  pltpu.sync_copy(z8_hbm, zbuf)

  base_r = sid * _RPT

  def zissue(i, carry):
    off = base_r + i * _ZR
    pltpu.async_copy(zbuf, mi_acc.at[pl.ds(off, _ZR)], sem_z)
    pltpu.async_copy(zbuf, mo_acc.at[pl.ds(off, _ZR)], sem_z)
    return carry
  lax.fori_loop(0, _NZC, zissue, 0)

  def zdrain(i, carry):
    pltpu.make_async_copy(zbuf, mi_acc.at[pl.ds(base_r, _ZR)], sem_z).wait()
    pltpu.make_async_copy(zbuf, mo_acc.at[pl.ds(base_r, _ZR)], sem_z).wait()
    return carry
  lax.fori_loop(0, _NZC, zdrain, 0)
  plsc.subcore_barrier()

  gcs = (gc0, gc1)
  grs = (gr0, gr1)
  mis = (mi0, mi1)
  mos = (mo0, mo1)
  sgs = (sem_g0, sem_g1)
  sss = (sem_s0, sem_s1)
  sis = (sem_i0, sem_i1)
  iota16 = lax.iota(jnp.int32, 16)
  tile_base_e = wid * _EPT
  tb_row = wid * _CHUNKS

  pltpu.sync_copy(col_hbm.at[tb_row], col_a.at[0])
  pltpu.sync_copy(row_hbm.at[tb_row], row_a.at[0])
  pltpu.async_copy(col_hbm.at[tb_row + 1], col_a.at[1], sem_i1)
  pltpu.async_copy(row_hbm.at[tb_row + 1], row_a.at[1], sem_i1)
  for _j in range(2):
    pltpu.async_copy(tc_hbm.at[col_a.at[0, _j]],
                     gc0.at[pl.ds(_j * _CS, _CS)], sem_g0)
    pltpu.async_copy(tr_hbm.at[row_a.at[0, _j]],
                     gr0.at[pl.ds(_j * _CS, _CS)], sem_g0)

  def outer(k4, carry):
    for u in range(4):
      k = k4 * 4 + u
      b = u % 2
      nb = 1 - b
      sl = u
      nsl = (u + 1) % 4
      fsl = (u + 2) % 4
      gcb = gcs[b]
      grb = grs[b]
      mib = mis[b]
      mob = mos[b]

      for _j in range(2):
        pltpu.make_async_copy(tc_hbm.at[col_a.at[sl, _j]],
                              gcb.at[pl.ds(_j * _CS, _CS)], sgs[b]).wait()
        pltpu.make_async_copy(tr_hbm.at[row_a.at[sl, _j]],
                              grb.at[pl.ds(_j * _CS, _CS)], sgs[b]).wait()

      @pl.when(k + 2 < _CHUNKS)
      def _issue_idx():
        pltpu.async_copy(col_hbm.at[tb_row + k + 2], col_a.at[fsl], sis[b])
        pltpu.async_copy(row_hbm.at[tb_row + k + 2], row_a.at[fsl], sis[b])

      @pl.when(k + 1 < _CHUNKS)
      def _issue_gather():
        pltpu.make_async_copy(col_hbm.at[tb_row + k + 1], col_a.at[nsl],
                              sis[nb]).wait()
        pltpu.make_async_copy(row_hbm.at[tb_row + k + 1], row_a.at[nsl],
                              sis[nb]).wait()
        for _j in range(2):
          pltpu.async_copy(tc_hbm.at[col_a.at[nsl, _j]],
                           gcs[nb].at[pl.ds(_j * _CS, _CS)], sgs[nb])
          pltpu.async_copy(tr_hbm.at[row_a.at[nsl, _j]],
                           grs[nb].at[pl.ds(_j * _CS, _CS)], sgs[nb])

      def grp(g, c2):
        rows = g * 16 + iota16
        s = be2k
        for j in range(8):
          a = plsc.load_gather(gcb, [rows, jnp.full((16,), j, jnp.int32)])
          bb = plsc.load_gather(grb, [rows, jnp.full((16,), j, jnp.int32)])
          q = jnp.exp(a + bb) + 1.0
          s = s - tw2[j] / q
        ev = 1.0 / (1.0 + jnp.exp(-s))
        gid = tile_base_e + k * _C + g * 16 + iota16
        ev = jnp.where(gid < _E, ev, 0.0)
        for f in range(_HID):
          pf = jnp.full((16,), 8 + f, jnp.int32)
          ff = jnp.full((16,), f, jnp.int32)
          xr = plsc.load_gather(grb, [rows, pf])
          plsc.store_scatter(mib, [rows, ff], ev * xr)
          xcv = plsc.load_gather(gcb, [rows, pf])
          plsc.store_scatter(mob, [rows, ff], ev * xcv)
        return c2
      lax.fori_loop(0, _C // 16, grp, 0)

      @pl.when(k >= 1)
      def _wait_prev_scatter():
        for _j in range(2):
          pltpu.make_async_copy(mis[nb].at[pl.ds(_j * _CS, _CS)],
                                mi_acc.at[col_a.at[sl, _j]], sss[nb]).wait()
          pltpu.make_async_copy(mos[nb].at[pl.ds(_j * _CS, _CS)],
                                mo_acc.at[row_a.at[sl, _j]], sss[nb]).wait()
      for _j in range(2):
        pltpu.async_copy(mib.at[pl.ds(_j * _CS, _CS)],
                         mi_acc.at[col_a.at[sl, _j]], sss[b], add=True)
        pltpu.async_copy(mob.at[pl.ds(_j * _CS, _CS)],
                         mo_acc.at[row_a.at[sl, _j]], sss[b], add=True)
    return carry
  lax.fori_loop(0, _CHUNKS // 4, outer, 0)

  for _j in range(2):
    pltpu.make_async_copy(mis[1].at[pl.ds(_j * _CS, _CS)],
                          mi_acc.at[col_a.at[0, _j]], sss[1]).wait()
    pltpu.make_async_copy(mos[1].at[pl.ds(_j * _CS, _CS)],
                          mo_acc.at[row_a.at[0, _j]], sss[1]).wait()
  plsc.subcore_barrier()

  def dissue(i, carry):
    off = base_r + i * _ZR
    pltpu.async_copy(mi_acc.at[pl.ds(off, _ZR)],
                     mi_out.at[cid, pl.ds(off, _ZR)], sem_z)
    pltpu.async_copy(mo_acc.at[pl.ds(off, _ZR)],
                     mo_out.at[cid, pl.ds(off, _ZR)], sem_z)
    return carry
  lax.fori_loop(0, _NZC, dissue, 0)

  def ddrain(i, carry):
    pltpu.make_async_copy(mi_acc.at[pl.ds(base_r, _ZR)],
                          mi_out.at[cid, pl.ds(base_r, _ZR)], sem_z).wait()
    pltpu.make_async_copy(mo_acc.at[pl.ds(base_r, _ZR)],
                          mo_out.at[cid, pl.ds(base_r, _ZR)], sem_z).wait()
    return carry
  lax.fori_loop(0, _NZC, ddrain, 0)


def _sc_msg(tc, tr, col, row, consts, z8):
  f = functools.partial(
      pl.kernel,
      out_type=(jax.ShapeDtypeStruct((_NC, _NP, _HID), jnp.float32),
                jax.ShapeDtypeStruct((_NC, _NP, _HID), jnp.float32)),
      mesh=_sc_mesh(),
      compiler_params=pltpu.CompilerParams(needs_layout_passes=False,
                                           use_tc_tiling_on_sc=False),
      scratch_types=[
          pltpu.VMEM_SHARED((_NP, _HID), jnp.float32),
          pltpu.VMEM_SHARED((_NP, _HID), jnp.float32),
          pltpu.VMEM((4, 2, _CS), jnp.int32),
          pltpu.VMEM((4, 2, _CS), jnp.int32),
          pltpu.VMEM((_C, 16), jnp.float32),
          pltpu.VMEM((_C, 16), jnp.float32),
          pltpu.VMEM((_C, 16), jnp.float32),
          pltpu.VMEM((_C, 16), jnp.float32),
          pltpu.VMEM((_C, _HID), jnp.float32),
          pltpu.VMEM((_C, _HID), jnp.float32),
          pltpu.VMEM((_C, _HID), jnp.float32),
          pltpu.VMEM((_C, _HID), jnp.float32),
          pltpu.VMEM((_ZR, _HID), jnp.float32),
          pltpu.VMEM((4, 16), jnp.float32),
          pltpu.SemaphoreType.DMA,
          pltpu.SemaphoreType.DMA,
          pltpu.SemaphoreType.DMA,
          pltpu.SemaphoreType.DMA,
          pltpu.SemaphoreType.DMA,
          pltpu.SemaphoreType.DMA,
          pltpu.SemaphoreType.DMA,
      ],
  )(_sc_msg_body)
  return f(tc, tr, col, row, consts, z8)


def _sc_final_body(tc_hbm, tr_hbm, col_hbm, row_hbm, consts_hbm, e_out,
                   col_a, row_a, gc0, gc1, gr0, gr1, ebuf, cbuf,
                   sem_g0, sem_g1, sem_i0, sem_i1):
  cid = lax.axis_index("c")
  sid = lax.axis_index("s")
  wid = cid * _NS + sid

  pltpu.sync_copy(consts_hbm, cbuf)
  tw2 = cbuf[1]
  be2k = cbuf[2]

  tb_row = wid * _CHUNKS

  gcs = (gc0, gc1)
  grs = (gr0, gr1)
  sgs = (sem_g0, sem_g1)
  sis = (sem_i0, sem_i1)
  iota16 = lax.iota(jnp.int32, 16)

  pltpu.sync_copy(col_hbm.at[tb_row], col_a.at[0])
  pltpu.sync_copy(row_hbm.at[tb_row], row_a.at[0])
  pltpu.async_copy(col_hbm.at[tb_row + 1], col_a.at[1], sem_i1)
  pltpu.async_copy(row_hbm.at[tb_row + 1], row_a.at[1], sem_i1)
  for _j in range(2):
    pltpu.async_copy(tc_hbm.at[col_a.at[0, _j]],
                     gc0.at[pl.ds(_j * _CS, _CS)], sem_g0)
    pltpu.async_copy(tr_hbm.at[row_a.at[0, _j]],
                     gr0.at[pl.ds(_j * _CS, _CS)], sem_g0)

  def outer(k4, carry):
    for u in range(4):
      k = k4 * 4 + u
      b = u % 2
      nb = 1 - b
      sl = u
      nsl = (u + 1) % 4
      fsl = (u + 2) % 4
      gcb = gcs[b]
      grb = grs[b]

      for _j in range(2):
        pltpu.make_async_copy(tc_hbm.at[col_a.at[sl, _j]],
                              gcb.at[pl.ds(_j * _CS, _CS)], sgs[b]).wait()
        pltpu.make_async_copy(tr_hbm.at[row_a.at[sl, _j]],
                              grb.at[pl.ds(_j * _CS, _CS)], sgs[b]).wait()

      @pl.when(k + 2 < _CHUNKS)
      def _issue_idx():
        pltpu.async_copy(col_hbm.at[tb_row + k + 2], col_a.at[fsl], sis[b])
        pltpu.async_copy(row_hbm.at[tb_row + k + 2], row_a.at[fsl], sis[b])

      @pl.when(k + 1 < _CHUNKS)
      def _issue_gather():
        pltpu.make_async_copy(col_hbm.at[tb_row + k + 1], col_a.at[nsl],
                              sis[nb]).wait()
        pltpu.make_async_copy(row_hbm.at[tb_row + k + 1], row_a.at[nsl],
                              sis[nb]).wait()
        for _j in range(2):
          pltpu.async_copy(tc_hbm.at[col_a.at[nsl, _j]],
                           gcs[nb].at[pl.ds(_j * _CS, _CS)], sgs[nb])
          pltpu.async_copy(tr_hbm.at[row_a.at[nsl, _j]],
                           grs[nb].at[pl.ds(_j * _CS, _CS)], sgs[nb])

      def grp(g, c2):
        rows = g * 16 + iota16
        s = be2k
        for j in range(8):
          a = plsc.load_gather(gcb, [rows, jnp.full((16,), j, jnp.int32)])
          bb = plsc.load_gather(grb, [rows, jnp.full((16,), j, jnp.int32)])
          q = jnp.exp(a + bb) + 1.0
          s = s - tw2[j] / q
        ev = 1.0 / (1.0 + jnp.exp(-s))
        off = pl.multiple_of(g * 16, 16)
        ebuf[k, pl.ds(off, 16)] = ev
        return c2
      lax.fori_loop(0, _C // 16, grp, 0)
    return carry
  lax.fori_loop(0, _CHUNKS // 4, outer, 0)

  pltpu.sync_copy(ebuf, e_out.at[pl.ds(tb_row, _CHUNKS)])


def _sc_final(tc, tr, col, row, consts):
  f = functools.partial(
      pl.kernel,
      out_type=jax.ShapeDtypeStruct((_ROWS_E, _C), jnp.float32),
      mesh=_sc_mesh(),
      compiler_params=pltpu.CompilerParams(needs_layout_passes=False,
                                           use_tc_tiling_on_sc=False),
      scratch_types=[
          pltpu.VMEM((4, 2, _CS), jnp.int32),
          pltpu.VMEM((4, 2, _CS), jnp.int32),
          pltpu.VMEM((_C, _HID), jnp.float32),
          pltpu.VMEM((_C, _HID), jnp.float32),
          pltpu.VMEM((_C, _HID), jnp.float32),
          pltpu.VMEM((_C, _HID), jnp.float32),
          pltpu.VMEM((_CHUNKS, _C), jnp.float32),
          pltpu.VMEM((4, 16), jnp.float32),
          pltpu.SemaphoreType.DMA,
          pltpu.SemaphoreType.DMA,
          pltpu.SemaphoreType.DMA,
          pltpu.SemaphoreType.DMA,
      ],
  )(_sc_final_body)
  return f(tc, tr, col, row, consts)


_BN = 2000


def _tables(xc, wa, wb, wn1a, wn1b, b1e):
  u = 2.0 * jnp.dot(xc, wa, preferred_element_type=jnp.float32) + b1e
  v = 2.0 * jnp.dot(xc, wb, preferred_element_type=jnp.float32) + b1e
  p1 = jnp.dot(xc, wn1a, preferred_element_type=jnp.float32)
  p2 = jnp.dot(xc, wn1b, preferred_element_type=jnp.float32)
  tcv = jnp.concatenate([u, p2], axis=1)
  trv = jnp.concatenate([v, p1], axis=1)
  return tcv, trv


def _tc_init_body(x_ref, w1, b1, wa, wb, wn1a, wn1b, b1e,
                  tc_ref, tr_ref, xc_ref):
  xb = x_ref[...]
  h = jnp.tanh(jnp.dot(xb, w1[...], preferred_element_type=jnp.float32)
               + b1[...])
  xc = jnp.concatenate([h, xb], axis=1)
  tcv, trv = _tables(xc, wa[...], wb[...], wn1a[...], wn1b[...], b1e[...])
  tc_ref[...] = tcv
  tr_ref[...] = trv
  z = jnp.zeros((xb.shape[0], 16 - _DIM), jnp.float32)
  xc_ref[...] = jnp.concatenate([xc, z], axis=1)


def _tc_init(x, w1, b1, wa, wb, wn1a, wn1b, b1e):
  wspec = pl.BlockSpec((_DIM, _HID), lambda i: (0, 0))
  bspec = pl.BlockSpec((1, _HID), lambda i: (0, 0))
  return pl.pallas_call(
      _tc_init_body,
      grid=(_N // _BN,),
      in_specs=[
          pl.BlockSpec((_BN, _IN), lambda i: (i, 0)),
          pl.BlockSpec((_IN, _HID), lambda i: (0, 0)),
          bspec, wspec, wspec, wspec, wspec, bspec,
      ],
      out_specs=[pl.BlockSpec((_BN, 16), lambda i: (i, 0))] * 3,
      out_shape=[jax.ShapeDtypeStruct((_N, 16), jnp.float32)] * 3,
  )(x, w1, b1, wa, wb, wn1a, wn1b, b1e)


def _tc_iter_body(mi2, mo2, xc_in, wn1c, bn1, wn2, bn2, wa, wb, wn1a, wn1b,
                  b1e, tc_ref, tr_ref, xc_ref):
  mi = mi2[0] + mi2[1]
  mo = mo2[0] + mo2[1]
  xc = xc_in[:, :_DIM]
  h1 = jnp.tanh(mi + mo
                + jnp.dot(xc, wn1c[...], preferred_element_type=jnp.float32)
                + bn1[...])
  hn = jnp.tanh(jnp.dot(h1, wn2[...], preferred_element_type=jnp.float32)
                + bn2[...])
  xcn = jnp.concatenate([hn, xc[:, _HID:_DIM]], axis=1)
  tcv, trv = _tables(xcn, wa[...], wb[...], wn1a[...], wn1b[...], b1e[...])
  tc_ref[...] = tcv
  tr_ref[...] = trv
  z = jnp.zeros((xcn.shape[0], 16 - _DIM), jnp.float32)
  xc_ref[...] = jnp.concatenate([xcn, z], axis=1)


def _tc_iter(mi2, mo2, xca, wn1c, bn1, wn2, bn2, wa, wb, wn1a, wn1b, b1e):
  wspec = pl.BlockSpec((_DIM, _HID), lambda i: (0, 0))
  bspec = pl.BlockSpec((1, _HID), lambda i: (0, 0))
  return pl.pallas_call(
      _tc_iter_body,
      grid=(_N // _BN,),
      in_specs=[
          pl.BlockSpec((_NC, _BN, _HID), lambda i: (0, i, 0)),
          pl.BlockSpec((_NC, _BN, _HID), lambda i: (0, i, 0)),
          pl.BlockSpec((_BN, 16), lambda i: (i, 0)),
          wspec, bspec,
          pl.BlockSpec((_HID, _HID), lambda i: (0, 0)),
          bspec, wspec, wspec, wspec, wspec, bspec,
      ],
      out_specs=[pl.BlockSpec((_BN, 16), lambda i: (i, 0))] * 3,
      out_shape=[jax.ShapeDtypeStruct((_N, 16), jnp.float32)] * 3,
  )(mi2, mo2, xca, wn1c, bn1, wn2, bn2, wa, wb, wn1a, wn1b, b1e)


def kernel(x, edge_index, W1, b1, We1, be1, We2, be2, Wn1, bn1, Wn2, bn2):
  row = edge_index[0].astype(jnp.int32)
  col = edge_index[1].astype(jnp.int32)
  pad = _EPAD - _E
  colp = jnp.concatenate([col, jnp.zeros((pad,), jnp.int32)])
  colp = colp.reshape(_ROWS_E, 2, _CS)
  rowp = jnp.concatenate([row, jnp.zeros((pad,), jnp.int32)])
  rowp = rowp.reshape(_ROWS_E, 2, _CS)

  wa = We1[:_DIM]
  wb = We1[_DIM:]
  wn1a = Wn1[:_DIM]
  wn1b = Wn1[_DIM:2 * _DIM]
  wn1c = Wn1[2 * _DIM:]
  zero8 = jnp.zeros((_HID,), jnp.float32)
  be1p = jnp.concatenate([be1, zero8])
  tw2 = jnp.concatenate([2.0 * We2[:, 0], zero8])
  be2k = jnp.full((16,), be2[0] + jnp.sum(We2[:, 0]), jnp.float32)
  consts = jnp.stack([be1p, tw2, be2k, jnp.zeros((16,), jnp.float32)])
  z8 = jnp.zeros((_ZR, _HID), jnp.float32)

  b1r = b1.reshape(1, _HID)
  bn1r = bn1.reshape(1, _HID)
  bn2r = bn2.reshape(1, _HID)
  b1e = be1.reshape(1, _HID)

  tc, tr, xca = _tc_init(x, W1, b1r, wa, wb, wn1a, wn1b, b1e)
  for _ in range(_NITER):
    mi2, mo2 = _sc_msg(tc, tr, colp, rowp, consts, z8)
    tc, tr, xca = _tc_iter(mi2, mo2, xca, wn1c, bn1r, Wn2, bn2r,
                           wa, wb, wn1a, wn1b, b1e)
  e = _sc_final(tc[:, :_HID], tr[:, :_HID], colp, rowp, consts)
  return e.reshape(_EPAD)[:_E]
